# baseline (speedup 1.0000x reference)
import functools

import jax
import jax.numpy as jnp
from jax import lax
from jax.experimental import pallas as pl
from jax.experimental.pallas import tpu as pltpu
from jax.experimental.pallas import tpu_sc as plsc

_EMBED_DIM = 32

_info = plsc.get_sparse_core_info()
_NC, _NS = _info.num_cores, _info.num_subcores
_NW = _NC * _NS


def _make_relayout(vocab: int, embed: int):
    vb = 512
    n_blocks = (vocab + vb - 1) // vb
    iters = (n_blocks + _NW - 1) // _NW + 1
    pairs = (iters + 1) // 2
    blk_elems = vb * embed
    mesh = plsc.VectorSubcoreMesh(core_axis_name="c", subcore_axis_name="s")

    @functools.partial(
        pl.kernel,
        mesh=mesh,
        compiler_params=pltpu.CompilerParams(needs_layout_passes=False),
        out_type=jax.ShapeDtypeStruct((n_blocks * blk_elems,), jnp.float32),
        scratch_types=[
            pltpu.VMEM((embed, vb), jnp.float32),
            pltpu.VMEM((embed, vb), jnp.float32),
            pltpu.VMEM((blk_elems,), jnp.float32),
            pltpu.VMEM((blk_elems,), jnp.float32),
            pltpu.SemaphoreType.DMA,
            pltpu.SemaphoreType.DMA,
            pltpu.SemaphoreType.DMA,
            pltpu.SemaphoreType.DMA,
        ],
    )
    def relayout_kernel(tt_hbm, out_hbm, tblk0, tblk1, rowblk0, rowblk1,
                        g0, g1, s0, s1):
        wid = lax.axis_index("s") * _NC + lax.axis_index("c")
        lanes = lax.iota(jnp.int32, 16)
        rot = [(lanes + d) % 16 for d in range(16)]
        rows_e0 = {e0: [r + e0 for r in rot] for e0 in (0, 16)}
        tblk = (tblk0, tblk1)
        rowblk = (rowblk0, rowblk1)
        gsem = (g0, g1)
        ssem = (s0, s1)
        def vstart_of(i):
            blk = jnp.minimum(wid + i * _NW, n_blocks - 1)
            return pl.multiple_of(blk * vb, vb)

        pltpu.async_copy(
            tt_hbm.at[:, pl.ds(vstart_of(0), vb)], tblk[0], g0
        )

        def pair_body(p, carry):
            for b in (0, 1):
                i = 2 * p + b
                pltpu.make_async_copy(
                    tt_hbm.at[:, pl.ds(0, vb)], tblk[b], gsem[b]
                ).wait()
                pltpu.async_copy(
                    tt_hbm.at[:, pl.ds(vstart_of(i + 1), vb)],
                    tblk[1 - b],
                    gsem[1 - b],
                )
                @pl.when(p >= 1)
                def _wait_store():
                    pltpu.make_async_copy(
                        rowblk[b], out_hbm.at[pl.ds(0, blk_elems)], ssem[b]
                    ).wait()

                def g_body(g, car):
                    cols = g * 16 + lanes
                    for e0 in (0, 16):
                        colbase = cols * embed + e0
                        for d in range(16):
                            v = plsc.load_gather(tblk[b], [rows_e0[e0][d], cols])
                            plsc.store_scatter(rowblk[b], [colbase + rot[d]], v)
                    return car

                lax.fori_loop(0, vb // 16, g_body, 0)

                pltpu.async_copy(
                    rowblk[b],
                    out_hbm.at[pl.ds(vstart_of(i) * embed, blk_elems)],
                    ssem[b],
                )
            return carry

        lax.fori_loop(0, pairs, pair_body, 0)

        pltpu.make_async_copy(
            tt_hbm.at[:, pl.ds(0, vb)], tblk[0], g0
        ).wait()
        pltpu.make_async_copy(
            rowblk[0], out_hbm.at[pl.ds(0, blk_elems)], s0
        ).wait()
        pltpu.make_async_copy(
            rowblk[1], out_hbm.at[pl.ds(0, blk_elems)], s1
        ).wait()

    return relayout_kernel


def _make_gather(batch: int, hist: int, chunk_b: int):
    b_per_w = batch // _NW
    assert b_per_w % chunk_b == 0
    n_chunks = b_per_w // chunk_b
    chunk = chunk_b * hist
    rows_per_w = b_per_w * hist
    n_rows = batch * hist
    mesh = plsc.VectorSubcoreMesh(core_axis_name="c", subcore_axis_name="s")

    @functools.partial(
        pl.kernel,
        mesh=mesh,
        compiler_params=pltpu.CompilerParams(
            use_tc_tiling_on_sc=False, needs_layout_passes=False
        ),
        out_type=jax.ShapeDtypeStruct((n_rows, _EMBED_DIM), jnp.float32),
        scratch_types=[
            pltpu.VMEM((b_per_w, hist), jnp.int32),
            pltpu.VMEM((rows_per_w,), jnp.int32),
            pltpu.VMEM((2, chunk, _EMBED_DIM), jnp.float32),
            pltpu.SemaphoreType.DMA,
            pltpu.SemaphoreType.DMA,
            pltpu.SemaphoreType.DMA,
            pltpu.SemaphoreType.DMA,
        ],
    )
    def gather_kernel(table_hbm, x_hbm, out_hbm, idx2d, idx_v, rows_v,
                      g0, g1, s0, s1):
        wid = lax.axis_index("s") * _NC + lax.axis_index("c")
        base = wid * rows_per_w
        pltpu.sync_copy(x_hbm.at[pl.ds(wid * b_per_w, b_per_w)], idx2d)

        lanes = lax.iota(jnp.int32, 16)

        def flat_body(j, carry):
            m = j * 16 + lanes
            v = plsc.load_gather(idx2d, [m // hist, m % hist])
            idx_v[pl.ds(j * 16, 16)] = v
            return carry

        lax.fori_loop(0, rows_per_w // 16, flat_body, 0)

        gsem = (g0, g1)
        ssem = (s0, s1)
        gathers = [None, None]
        stores = [None, None]
        gathers[0] = pltpu.async_copy(
            table_hbm.at[idx_v.at[pl.ds(0, chunk)]], rows_v.at[0], g0
        )
        for i in range(n_chunks):
            b = i % 2
            nb = (i + 1) % 2
            if i + 1 < n_chunks:
                if stores[nb] is not None:
                    stores[nb].wait()
                gathers[nb] = pltpu.async_copy(
                    table_hbm.at[idx_v.at[pl.ds((i + 1) * chunk, chunk)]],
                    rows_v.at[nb],
                    gsem[nb],
                )
            gathers[b].wait()
            stores[b] = pltpu.async_copy(
                rows_v.at[b], out_hbm.at[pl.ds(base + i * chunk, chunk)], ssem[b]
            )
        stores[(n_chunks - 1) % 2].wait()
        if n_chunks >= 2:
            stores[(n_chunks - 2) % 2].wait()

    return gather_kernel


def kernel(x, table):
    batch, hist = x.shape
    vocab, embed = table.shape
    table_lin = _make_relayout(vocab, embed)(table.T)
    vocab_pad = table_lin.shape[0] // embed
    out = _make_gather(batch, hist, 64)(
        table_lin.reshape(vocab_pad, embed), x.astype(jnp.int32)
    )
    return out.reshape(batch, hist, _EMBED_DIM)


# speedup vs baseline: 1.0342x; 1.0342x over previous
import functools

import jax
import jax.numpy as jnp
from jax import lax
from jax.experimental import pallas as pl
from jax.experimental.pallas import tpu as pltpu
from jax.experimental.pallas import tpu_sc as plsc

_EMBED_DIM = 32

_info = plsc.get_sparse_core_info()
_NC, _NS = _info.num_cores, _info.num_subcores
_NW = _NC * _NS


def _make_relayout(vocab: int, embed: int):
    vb = 256
    n_blocks = (vocab + vb - 1) // vb
    iters = (n_blocks + _NW - 1) // _NW + 1
    pairs = (iters + 1) // 2
    blk_elems = vb * embed
    mesh = plsc.VectorSubcoreMesh(core_axis_name="c", subcore_axis_name="s")

    @functools.partial(
        pl.kernel,
        mesh=mesh,
        compiler_params=pltpu.CompilerParams(needs_layout_passes=False),
        out_type=jax.ShapeDtypeStruct((n_blocks * blk_elems,), jnp.float32),
        scratch_types=[
            pltpu.VMEM((embed, vb), jnp.float32),
            pltpu.VMEM((embed, vb), jnp.float32),
            pltpu.VMEM((blk_elems,), jnp.float32),
            pltpu.VMEM((blk_elems,), jnp.float32),
            pltpu.SemaphoreType.DMA,
            pltpu.SemaphoreType.DMA,
            pltpu.SemaphoreType.DMA,
            pltpu.SemaphoreType.DMA,
        ],
    )
    def relayout_kernel(tt_hbm, out_hbm, tblk0, tblk1, rowblk0, rowblk1,
                        g0, g1, s0, s1):
        wid = lax.axis_index("s") * _NC + lax.axis_index("c")
        lanes = lax.iota(jnp.int32, 16)
        rot = [(lanes + d) % 16 for d in range(16)]
        rows_e0 = {e0: [r + e0 for r in rot] for e0 in (0, 16)}
        tblk = (tblk0, tblk1)
        rowblk = (rowblk0, rowblk1)
        gsem = (g0, g1)
        ssem = (s0, s1)
        def vstart_of(i):
            blk = jnp.minimum(wid + i * _NW, n_blocks - 1)
            return pl.multiple_of(blk * vb, vb)

        pltpu.async_copy(
            tt_hbm.at[:, pl.ds(vstart_of(0), vb)], tblk[0], g0
        )

        def pair_body(p, carry):
            for b in (0, 1):
                i = 2 * p + b
                pltpu.make_async_copy(
                    tt_hbm.at[:, pl.ds(0, vb)], tblk[b], gsem[b]
                ).wait()
                pltpu.async_copy(
                    tt_hbm.at[:, pl.ds(vstart_of(i + 1), vb)],
                    tblk[1 - b],
                    gsem[1 - b],
                )
                @pl.when(p >= 1)
                def _wait_store():
                    pltpu.make_async_copy(
                        rowblk[b], out_hbm.at[pl.ds(0, blk_elems)], ssem[b]
                    ).wait()

                def g_body(g, car):
                    cols = g * 16 + lanes
                    for e0 in (0, 16):
                        colbase = cols * embed + e0
                        for d in range(16):
                            v = plsc.load_gather(tblk[b], [rows_e0[e0][d], cols])
                            plsc.store_scatter(rowblk[b], [colbase + rot[d]], v)
                    return car

                lax.fori_loop(0, vb // 16, g_body, 0)

                pltpu.async_copy(
                    rowblk[b],
                    out_hbm.at[pl.ds(vstart_of(i) * embed, blk_elems)],
                    ssem[b],
                )
            return carry

        lax.fori_loop(0, pairs, pair_body, 0)

        pltpu.make_async_copy(
            tt_hbm.at[:, pl.ds(0, vb)], tblk[0], g0
        ).wait()
        pltpu.make_async_copy(
            rowblk[0], out_hbm.at[pl.ds(0, blk_elems)], s0
        ).wait()
        pltpu.make_async_copy(
            rowblk[1], out_hbm.at[pl.ds(0, blk_elems)], s1
        ).wait()

    return relayout_kernel


def _make_gather(batch: int, hist: int, chunk_b: int):
    b_per_w = batch // _NW
    assert b_per_w % chunk_b == 0
    n_chunks = b_per_w // chunk_b
    chunk = chunk_b * hist
    rows_per_w = b_per_w * hist
    n_rows = batch * hist
    mesh = plsc.VectorSubcoreMesh(core_axis_name="c", subcore_axis_name="s")

    @functools.partial(
        pl.kernel,
        mesh=mesh,
        compiler_params=pltpu.CompilerParams(
            use_tc_tiling_on_sc=False, needs_layout_passes=False
        ),
        out_type=jax.ShapeDtypeStruct((n_rows, _EMBED_DIM), jnp.float32),
        scratch_types=[
            pltpu.VMEM((b_per_w, hist), jnp.int32),
            pltpu.VMEM((rows_per_w,), jnp.int32),
            pltpu.VMEM((2, chunk, _EMBED_DIM), jnp.float32),
            pltpu.SemaphoreType.DMA,
            pltpu.SemaphoreType.DMA,
            pltpu.SemaphoreType.DMA,
            pltpu.SemaphoreType.DMA,
        ],
    )
    def gather_kernel(table_hbm, x_hbm, out_hbm, idx2d, idx_v, rows_v,
                      g0, g1, s0, s1):
        wid = lax.axis_index("s") * _NC + lax.axis_index("c")
        base = wid * rows_per_w
        pltpu.sync_copy(x_hbm.at[pl.ds(wid * b_per_w, b_per_w)], idx2d)

        lanes = lax.iota(jnp.int32, 16)

        def flat_body(j, carry):
            m = j * 16 + lanes
            v = plsc.load_gather(idx2d, [m // hist, m % hist])
            idx_v[pl.ds(j * 16, 16)] = v
            return carry

        lax.fori_loop(0, rows_per_w // 16, flat_body, 0)

        gsem = (g0, g1)
        ssem = (s0, s1)
        gathers = [None, None]
        stores = [None, None]
        gathers[0] = pltpu.async_copy(
            table_hbm.at[idx_v.at[pl.ds(0, chunk)]], rows_v.at[0], g0
        )
        for i in range(n_chunks):
            b = i % 2
            nb = (i + 1) % 2
            if i + 1 < n_chunks:
                if stores[nb] is not None:
                    stores[nb].wait()
                gathers[nb] = pltpu.async_copy(
                    table_hbm.at[idx_v.at[pl.ds((i + 1) * chunk, chunk)]],
                    rows_v.at[nb],
                    gsem[nb],
                )
            gathers[b].wait()
            stores[b] = pltpu.async_copy(
                rows_v.at[b], out_hbm.at[pl.ds(base + i * chunk, chunk)], ssem[b]
            )
        stores[(n_chunks - 1) % 2].wait()
        if n_chunks >= 2:
            stores[(n_chunks - 2) % 2].wait()

    return gather_kernel


def kernel(x, table):
    batch, hist = x.shape
    vocab, embed = table.shape
    table_lin = _make_relayout(vocab, embed)(table.T)
    vocab_pad = table_lin.shape[0] // embed
    out = _make_gather(batch, hist, 64)(
        table_lin.reshape(vocab_pad, embed), x.astype(jnp.int32)
    )
    return out.reshape(batch, hist, _EMBED_DIM)
